# baseline (speedup 1.0000x reference)
import functools

import jax
import jax.numpy as jnp
from jax import lax
from jax.experimental import pallas as pl
from jax.experimental.pallas import tpu as pltpu
from jax.experimental.pallas import tpu_sc as plsc

_VOCAB = 100000
_D = 512
_L = 200
_B = 1024
_N = _B * _L
_NC = 2
_NS = 16
_NW = _NC * _NS
_LC = 40
_NCHUNK = _L // _LC
_BATCH_PER_W = _B // _NW
_PAIRS = _BATCH_PER_W // 2
_IDXROWS_PER_W = _BATCH_PER_W * _NCHUNK
_LANES = 16
_NRING = 2
_SLOTS = _PAIRS + _NRING


def _pos_encoding():
    even_i = jnp.arange(0, _D, 2).astype(jnp.float32)
    denom = jnp.power(10000.0, even_i / _D)
    position = jnp.arange(_L).reshape(_L, 1).astype(jnp.float32)
    even_pe = jnp.sin(position / denom)
    odd_pe = jnp.cos(position / denom)
    return jnp.stack([even_pe, odd_pe], axis=2).reshape(_L, _D)


def _sc_embed(table, tokens2d, pos):
    mesh = plsc.VectorSubcoreMesh(core_axis_name="c", subcore_axis_name="s")

    @functools.partial(
        pl.kernel,
        mesh=mesh,
        out_type=jax.ShapeDtypeStruct((_B, _L, _D), jnp.float32),
        scratch_types=[
            pltpu.VMEM((_IDXROWS_PER_W, _LC), jnp.int32),
            pltpu.VMEM((_LC, _D), jnp.float32),
            pltpu.VMEM((_NRING, 2, _LC, _D), jnp.float32),
        ]
        + [pltpu.SemaphoreType.DMA for _ in range(3 * _NRING)],
    )
    def k(table_hbm, tok_hbm, pos_hbm, out_hbm, idx_v, pos_v, ring, *sems):
        gsems0 = sems[0:_NRING]
        gsems1 = sems[_NRING:2 * _NRING]
        wsems = sems[2 * _NRING:3 * _NRING]
        wid = lax.axis_index("s") * _NC + lax.axis_index("c")
        base_b = wid * _BATCH_PER_W
        pltpu.sync_copy(
            tok_hbm.at[pl.ds(wid * _IDXROWS_PER_W, _IDXROWS_PER_W)], idx_v
        )

        def wait_write(p):
            pltpu.make_async_copy(
                ring.at[p],
                out_hbm.at[pl.ds(0, 2), pl.ds(0, _LC), pl.ds(0, _D)],
                wsems[p],
            ).wait()

        def wait_gathers(p):
            pltpu.make_async_copy(
                table_hbm.at[pl.ds(0, _LC)], ring.at[p, 0], gsems0[p]
            ).wait()
            pltpu.make_async_copy(
                table_hbm.at[pl.ds(0, _LC)], ring.at[p, 1], gsems1[p]
            ).wait()

        def chunk_body(c, carry):
            pltpu.sync_copy(pos_hbm.at[pl.ds(c * _LC, _LC)], pos_v)

            def slot_group(j, carry2):
                for p in range(_NRING):
                    t = j * _NRING + p
                    v = t - 1
                    pv = (p + _NRING - 1) % _NRING

                    @pl.when(t < _PAIRS)
                    def _issue():
                        @pl.when(c * _PAIRS + t >= _NRING)
                        def _drain():
                            wait_write(p)

                        row = (2 * t) * _NCHUNK + c
                        pltpu.async_copy(
                            table_hbm.at[idx_v.at[row]], ring.at[p, 0],
                            gsems0[p],
                        )
                        row2 = (2 * t + 1) * _NCHUNK + c
                        pltpu.async_copy(
                            table_hbm.at[idx_v.at[row2]], ring.at[p, 1],
                            gsems1[p],
                        )

                    @pl.when(jnp.logical_and(v >= 0, v < _PAIRS))
                    def _consume():
                        wait_gathers(pv)

                        def add_body(r, carry3):
                            for s in range(2):
                                for kk in range(_D // _LANES):
                                    sl = pl.ds(kk * _LANES, _LANES)
                                    plsc.addupdate(
                                        ring.at[pv, s, r, sl], pos_v[r, sl]
                                    )
                            return carry3

                        lax.fori_loop(0, _LC, add_body, 0)
                        pltpu.async_copy(
                            ring.at[pv],
                            out_hbm.at[
                                pl.ds(base_b + 2 * v, 2),
                                pl.ds(c * _LC, _LC),
                                pl.ds(0, _D),
                            ],
                            wsems[pv],
                        )
                return carry2

            lax.fori_loop(0, _SLOTS // _NRING, slot_group, 0)
            return carry

        lax.fori_loop(0, _NCHUNK, chunk_body, 0)
        for p in range(_NRING):
            wait_write(p)

    return k(table, tokens2d, pos)


def kernel(tokens, table):
    pos = _pos_encoding()
    tokens2d = tokens.reshape(_N // _LC, _LC).astype(jnp.int32)
    return _sc_embed(table, tokens2d, pos)


# speedup vs baseline: 1.5715x; 1.5715x over previous
import functools

import jax
import jax.numpy as jnp
from jax import lax
from jax.experimental import pallas as pl
from jax.experimental.pallas import tpu as pltpu
from jax.experimental.pallas import tpu_sc as plsc

_VOCAB = 100000
_D = 512
_L = 200
_B = 1024
_N = _B * _L
_NC = 2
_NS = 16
_NW = _NC * _NS
_ROWS_PER_W = _N // _NW
_LC = 40
_SPLIT = 24
_NCHUNK = _L // _LC
_BATCH_PER_W = _B // _NW
_IDXROWS_PER_W = _ROWS_PER_W // _LC
_LANES = 16
_NBUF = 4
_LAG = 2
_SLOTS = _BATCH_PER_W + _LAG + 2


def _pos_encoding():
    even_i = jnp.arange(0, _D, 2).astype(jnp.float32)
    denom = jnp.power(10000.0, even_i / _D)
    position = jnp.arange(_L).reshape(_L, 1).astype(jnp.float32)
    even_pe = jnp.sin(position / denom)
    odd_pe = jnp.cos(position / denom)
    return jnp.stack([even_pe, odd_pe], axis=2).reshape(_L, _D)


def _sc_embed(table, tokens2d, pos):
    mesh = plsc.VectorSubcoreMesh(core_axis_name="c", subcore_axis_name="s")

    @functools.partial(
        pl.kernel,
        mesh=mesh,
        out_type=jax.ShapeDtypeStruct((_N, _D), jnp.float32),
        scratch_types=[
            pltpu.VMEM((_IDXROWS_PER_W, _LC), jnp.int32),
            pltpu.VMEM((_LC, _D), jnp.float32),
        ]
        + [pltpu.VMEM((_LC, _D), jnp.float32) for _ in range(_NBUF)]
        + [pltpu.SemaphoreType.DMA for _ in range(4 * _NBUF)],
    )
    def k(table_hbm, tok_hbm, pos_hbm, out_hbm, idx_v, pos_v, *refs):
        rows = refs[:_NBUF]
        gsa = refs[_NBUF:2 * _NBUF]
        gsb = refs[2 * _NBUF:3 * _NBUF]
        wsa = refs[3 * _NBUF:4 * _NBUF]
        wsb = refs[4 * _NBUF:5 * _NBUF]
        wid = lax.axis_index("s") * _NC + lax.axis_index("c")
        base = wid * _ROWS_PER_W
        pltpu.sync_copy(
            tok_hbm.at[pl.ds(wid * _IDXROWS_PER_W, _IDXROWS_PER_W)], idx_v
        )

        _NA, _NB = _SPLIT, _LC - _SPLIT

        def wait_sem(sem, nrows, buf):
            pltpu.make_async_copy(
                table_hbm.at[pl.ds(0, nrows)], buf.at[pl.ds(0, nrows)], sem
            ).wait()

        def chunk_body(c, carry):
            pltpu.sync_copy(pos_hbm.at[pl.ds(c * _LC, _LC)], pos_v)

            def slot_group(j, carry2):
                for b in range(_NBUF):
                    i = j * _NBUF + b
                    g = i
                    u = i - _LAG
                    bu = (b + _NBUF - _LAG) % _NBUF

                    @pl.when(g < _BATCH_PER_W)
                    def _issue():
                        @pl.when(c * _BATCH_PER_W + g >= _NBUF)
                        def _drain():
                            wait_sem(wsa[b], _NA, rows[b])
                            wait_sem(wsb[b], _NB, rows[b])

                        row = g * _NCHUNK + c
                        pltpu.async_copy(
                            table_hbm.at[idx_v.at[row, pl.ds(0, _NA)]],
                            rows[b].at[pl.ds(0, _NA)], gsa[b],
                        )
                        pltpu.async_copy(
                            table_hbm.at[idx_v.at[row, pl.ds(_NA, _NB)]],
                            rows[b].at[pl.ds(_NA, _NB)], gsb[b],
                        )

                    @pl.when(jnp.logical_and(u >= 0, u < _BATCH_PER_W))
                    def _consume():
                        rv = rows[bu]
                        out_off = base + u * _L + c * _LC

                        def add_body(r, carry3):
                            for kk in range(_D // _LANES):
                                sl = pl.ds(kk * _LANES, _LANES)
                                plsc.addupdate(rv.at[r, sl], pos_v[r, sl])
                            return carry3

                        wait_sem(gsa[bu], _NA, rv)
                        lax.fori_loop(0, _NA, add_body, 0)
                        pltpu.async_copy(
                            rv.at[pl.ds(0, _NA)],
                            out_hbm.at[pl.ds(out_off, _NA)], wsa[bu],
                        )
                        wait_sem(gsb[bu], _NB, rv)
                        lax.fori_loop(_NA, _LC, add_body, 0)
                        pltpu.async_copy(
                            rv.at[pl.ds(_NA, _NB)],
                            out_hbm.at[pl.ds(out_off + _NA, _NB)], wsb[bu],
                        )
                return carry2

            lax.fori_loop(0, _SLOTS // _NBUF, slot_group, 0)
            return carry

        lax.fori_loop(0, _NCHUNK, chunk_body, 0)
        for b in range(_NBUF):
            wait_sem(wsa[b], _NA, rows[b])
            wait_sem(wsb[b], _NB, rows[b])

    return k(table, tokens2d, pos)


def kernel(tokens, table):
    pos = _pos_encoding()
    tokens2d = tokens.reshape(_N // _LC, _LC).astype(jnp.int32)
    out = _sc_embed(table, tokens2d, pos)
    return out.reshape(_B, _L, _D)
